# baseline (speedup 1.0000x reference)
import functools

import jax
import jax.numpy as jnp
from jax import lax
from jax.experimental import pallas as pl
from jax.experimental.pallas import tpu as pltpu
from jax.experimental.pallas import tpu_sc as plsc

F32 = jnp.float32
BF16 = jnp.bfloat16

_NC, _NS = 2, 16
_NW = _NC * _NS


def _sc_gather(table, idx, chunk):
    v, d = table.shape
    b = idx.shape[0]
    b_per_w = b // _NW
    assert b % _NW == 0 and b_per_w % chunk == 0 and chunk % 8 == 0
    n_chunks = b_per_w // chunk
    mesh = plsc.VectorSubcoreMesh(core_axis_name="c", subcore_axis_name="s")

    @functools.partial(
        pl.kernel,
        out_type=jax.ShapeDtypeStruct((b, d), table.dtype),
        mesh=mesh,
        scratch_types=[
            pltpu.VMEM((3, chunk), jnp.int32),
            pltpu.VMEM((3 * chunk, d), table.dtype),
            pltpu.SemaphoreType.DMA((3,)),
            pltpu.SemaphoreType.DMA((3,)),
        ],
    )
    def gather_kernel(table_hbm, idx_hbm, out_hbm, idx_v, rows_v, gsem, wsem):
        wid = lax.axis_index("s") * _NC + lax.axis_index("c")
        base = wid * b_per_w
        gathers = [None, None, None]
        writes = [None, None, None]
        for c in range(n_chunks):
            j = c % 3
            if writes[j] is not None:
                writes[j].wait()
            off = base + c * chunk
            pltpu.sync_copy(idx_hbm.at[pl.ds(off, chunk)], idx_v.at[j])
            gathers[j] = pltpu.async_copy(
                table_hbm.at[idx_v.at[j]], rows_v.at[pl.ds(j * chunk, chunk)],
                gsem.at[j])
            if c >= 1:
                pj = (c - 1) % 3
                gathers[pj].wait()
                poff = base + (c - 1) * chunk
                writes[pj] = pltpu.async_copy(
                    rows_v.at[pl.ds(pj * chunk, chunk)],
                    out_hbm.at[pl.ds(poff, chunk)], wsem.at[pj])
        lj = (n_chunks - 1) % 3
        gathers[lj].wait()
        writes[lj] = pltpu.async_copy(
            rows_v.at[pl.ds(lj * chunk, chunk)],
            out_hbm.at[pl.ds(base + (n_chunks - 1) * chunk, chunk)],
            wsem.at[lj])
        for w in writes:
            if w is not None:
                w.wait()

    return gather_kernel(table, idx)


def _agg_layer(dif, gathered, w, relu, bm, bk, out_dtype):
    m_total, s_total = dif.shape
    d = gathered.shape[1]
    grid = (m_total // bm, s_total // bk)
    n_k = grid[1]
    dst_block0 = s_total // bm

    def body(dif_ref, src_ref, dst_ref, wt_ref, wb_ref, o_ref, acc_ref):
        k = pl.program_id(1)

        @pl.when(k == 0)
        def _():
            acc_ref[...] = jnp.zeros_like(acc_ref)

        acc_ref[...] += jnp.dot(dif_ref[...].astype(BF16),
                                src_ref[...].astype(BF16),
                                preferred_element_type=F32)

        @pl.when(k == n_k - 1)
        def _():
            o = (jnp.dot(acc_ref[...].astype(BF16), wt_ref[...].astype(BF16),
                         preferred_element_type=F32)
                 + jnp.dot(dst_ref[...].astype(BF16), wb_ref[...].astype(BF16),
                           preferred_element_type=F32))
            if relu:
                o = jnp.maximum(o, 0.0)
            o_ref[...] = o.astype(out_dtype)

    return pl.pallas_call(
        body,
        grid=grid,
        in_specs=[
            pl.BlockSpec((bm, bk), lambda m, k: (m, k)),
            pl.BlockSpec((bk, d), lambda m, k: (k, 0)),
            pl.BlockSpec((bm, d), lambda m, k: (dst_block0 + m, 0)),
            pl.BlockSpec((d, d), lambda m, k: (0, 0)),
            pl.BlockSpec((d, d), lambda m, k: (1, 0)),
        ],
        out_specs=pl.BlockSpec((bm, d), lambda m, k: (m, 0)),
        out_shape=jax.ShapeDtypeStruct((m_total, d), out_dtype),
        scratch_shapes=[pltpu.VMEM((bm, d), F32)],
        compiler_params=pltpu.CompilerParams(
            dimension_semantics=("parallel", "arbitrary")),
    )(dif, gathered, gathered, w, w)


def _mlp_head(x0, x1, W1, b1, W2, b2, W3, b3, W4, b4, W5, b5):
    n = x0.shape[0]
    d = x0.shape[1]

    def body(x0_ref, x1_ref, w1t_ref, w1b_ref, b1_ref, w2_ref, b2_ref,
             w3_ref, b3_ref, w4_ref, b4_ref, w5_ref, b5_ref, o_ref):
        z = (jnp.dot(x0_ref[...], w1t_ref[...], preferred_element_type=F32)
             + jnp.dot(x1_ref[...], w1b_ref[...], preferred_element_type=F32)
             + b1_ref[...])
        z = jnp.maximum(z, 0.0)
        z = jnp.maximum(jnp.dot(z, w2_ref[...], preferred_element_type=F32)
                        + b2_ref[...], 0.0)
        z = jnp.maximum(jnp.dot(z, w3_ref[...], preferred_element_type=F32)
                        + b3_ref[...], 0.0)
        z = jnp.maximum(jnp.dot(z, w4_ref[...], preferred_element_type=F32)
                        + b4_ref[...], 0.0)
        o_ref[...] = (jnp.dot(z, w5_ref[...], preferred_element_type=F32)
                      + b5_ref[...])

    specs = [
        pl.BlockSpec((n, d), lambda i: (0, 0)),
        pl.BlockSpec((n, d), lambda i: (0, 0)),
        pl.BlockSpec((d, 128), lambda i: (0, 0)),
        pl.BlockSpec((d, 128), lambda i: (1, 0)),
        pl.BlockSpec((1, 128), lambda i: (0, 0)),
        pl.BlockSpec((128, 64), lambda i: (0, 0)),
        pl.BlockSpec((1, 64), lambda i: (0, 0)),
        pl.BlockSpec((64, 32), lambda i: (0, 0)),
        pl.BlockSpec((1, 32), lambda i: (0, 0)),
        pl.BlockSpec((32, 8), lambda i: (0, 0)),
        pl.BlockSpec((1, 8), lambda i: (0, 0)),
        pl.BlockSpec((8, 1), lambda i: (0, 0)),
        pl.BlockSpec((1, 1), lambda i: (0, 0)),
    ]
    return pl.pallas_call(
        body,
        grid=(1,),
        in_specs=specs,
        out_specs=pl.BlockSpec((n, 1), lambda i: (0, 0)),
        out_shape=jax.ShapeDtypeStruct((n, 1), F32),
    )(x0, x1, W1, W1, b1.reshape(1, -1), W2, b2.reshape(1, -1),
      W3, b3.reshape(1, -1), W4, b4.reshape(1, -1), W5, b5.reshape(1, -1))


def _branch_layer1(feats, src_nodes, s2s, s2d, dif, w_agg1):
    idx = jnp.concatenate([jnp.take(src_nodes, s2s, axis=0),
                           jnp.take(src_nodes, s2d, axis=0)])
    g = _sc_gather(feats, idx, chunk=64)
    return _agg_layer(dif, g, w_agg1, relu=True, bm=512, bk=2048,
                      out_dtype=F32)


def _branch_layer2(h1, s2s, s2d, dif, w_agg2):
    idx = jnp.concatenate([s2s, s2d])
    g = _sc_gather(h1, idx, chunk=32)
    return _agg_layer(dif, g, w_agg2, relu=False, bm=1024, bk=2048,
                      out_dtype=F32)


def kernel(feats, src_nodes0, dstsrc2src0_1, dstsrc2src0_2, dstsrc2dst0_1,
           dstsrc2dst0_2, dif_mat0_1, dif_mat0_2, src_nodes1, dstsrc2src1_1,
           dstsrc2src1_2, dstsrc2dst1_1, dstsrc2dst1_2, dif_mat1_1,
           dif_mat1_2, w_agg1, w_agg2, W1, b1, W2, b2, W3, b3, W4, b4, W5,
           b5):
    h1_0 = _branch_layer1(feats, src_nodes0, dstsrc2src0_2, dstsrc2dst0_2,
                          dif_mat0_2, w_agg1)
    h1_1 = _branch_layer1(feats, src_nodes1, dstsrc2src1_2, dstsrc2dst1_2,
                          dif_mat1_2, w_agg1)
    x = _branch_layer2(h1_0, dstsrc2src0_1, dstsrc2dst0_1, dif_mat0_1, w_agg2)
    y = _branch_layer2(h1_1, dstsrc2src1_1, dstsrc2dst1_1, dif_mat1_1, w_agg2)
    return _mlp_head(x, y, W1, b1, W2, b2, W3, b3, W4, b4, W5, b5)


# speedup vs baseline: 1.1237x; 1.1237x over previous
import functools

import jax
import jax.numpy as jnp
from jax import lax
from jax.experimental import pallas as pl
from jax.experimental.pallas import tpu as pltpu
from jax.experimental.pallas import tpu_sc as plsc

F32 = jnp.float32
BF16 = jnp.bfloat16

_NC, _NS = 2, 16
_NW = _NC * _NS


def _sc_gather(table, idx, chunk):
    v, d = table.shape
    b = idx.shape[0]
    b_per_w = b // _NW
    assert b % _NW == 0 and b_per_w % chunk == 0 and chunk % 8 == 0
    n_chunks = b_per_w // chunk
    mesh = plsc.VectorSubcoreMesh(core_axis_name="c", subcore_axis_name="s")

    @functools.partial(
        pl.kernel,
        out_type=jax.ShapeDtypeStruct((b, d), table.dtype),
        mesh=mesh,
        scratch_types=[
            pltpu.VMEM((3, chunk), jnp.int32),
            pltpu.VMEM((3 * chunk, d), table.dtype),
            pltpu.SemaphoreType.DMA((3,)),
            pltpu.SemaphoreType.DMA((3,)),
        ],
    )
    def gather_kernel(table_hbm, idx_hbm, out_hbm, idx_v, rows_v, gsem, wsem):
        wid = lax.axis_index("s") * _NC + lax.axis_index("c")
        base = wid * b_per_w
        gathers = [None, None, None]
        writes = [None, None, None]
        for c in range(n_chunks):
            j = c % 3
            if writes[j] is not None:
                writes[j].wait()
            off = base + c * chunk
            pltpu.sync_copy(idx_hbm.at[pl.ds(off, chunk)], idx_v.at[j])
            gathers[j] = pltpu.async_copy(
                table_hbm.at[idx_v.at[j]], rows_v.at[pl.ds(j * chunk, chunk)],
                gsem.at[j])
            if c >= 1:
                pj = (c - 1) % 3
                gathers[pj].wait()
                poff = base + (c - 1) * chunk
                writes[pj] = pltpu.async_copy(
                    rows_v.at[pl.ds(pj * chunk, chunk)],
                    out_hbm.at[pl.ds(poff, chunk)], wsem.at[pj])
        lj = (n_chunks - 1) % 3
        gathers[lj].wait()
        writes[lj] = pltpu.async_copy(
            rows_v.at[pl.ds(lj * chunk, chunk)],
            out_hbm.at[pl.ds(base + (n_chunks - 1) * chunk, chunk)],
            wsem.at[lj])
        for w in writes:
            if w is not None:
                w.wait()

    return gather_kernel(table, idx)


def _agg_layer(dif, gathered, w, relu, bm, bk, out_dtype):
    m_total, s_total = dif.shape
    d = gathered.shape[1]
    grid = (m_total // bm, s_total // bk)
    n_k = grid[1]
    dst_block0 = s_total // bm

    def body(dif_ref, src_ref, dst_ref, wt_ref, wb_ref, o_ref, acc_ref,
             srcb_ref):
        m = pl.program_id(0)
        k = pl.program_id(1)

        @pl.when(m == 0)
        def _():
            srcb_ref[pl.ds(k * bk, bk), :] = (
                src_ref[pl.ds(k * bk, bk), :].astype(BF16))

        @pl.when(k == 0)
        def _():
            acc_ref[...] = jnp.zeros_like(acc_ref)

        acc_ref[...] += jnp.dot(dif_ref[...].astype(BF16),
                                srcb_ref[pl.ds(k * bk, bk), :],
                                preferred_element_type=F32)

        @pl.when(k == n_k - 1)
        def _():
            o = (jnp.dot(acc_ref[...].astype(BF16), wt_ref[...].astype(BF16),
                         preferred_element_type=F32)
                 + jnp.dot(dst_ref[...].astype(BF16), wb_ref[...].astype(BF16),
                           preferred_element_type=F32))
            if relu:
                o = jnp.maximum(o, 0.0)
            o_ref[...] = o.astype(out_dtype)

    return pl.pallas_call(
        body,
        grid=grid,
        in_specs=[
            pl.BlockSpec((bm, bk), lambda m, k: (m, k)),
            pl.BlockSpec((s_total, d), lambda m, k: (0, 0)),
            pl.BlockSpec((bm, d), lambda m, k: (dst_block0 + m, 0)),
            pl.BlockSpec((d, d), lambda m, k: (0, 0)),
            pl.BlockSpec((d, d), lambda m, k: (1, 0)),
        ],
        out_specs=pl.BlockSpec((bm, d), lambda m, k: (m, 0)),
        out_shape=jax.ShapeDtypeStruct((m_total, d), out_dtype),
        scratch_shapes=[pltpu.VMEM((bm, d), F32),
                        pltpu.VMEM((s_total, d), BF16)],
        compiler_params=pltpu.CompilerParams(
            dimension_semantics=("parallel", "arbitrary")),
    )(dif, gathered, gathered, w, w)


def _mlp_head(x0, x1, W1, b1, W2, b2, W3, b3, W4, b4, W5, b5):
    n = x0.shape[0]
    d = x0.shape[1]

    def body(x0_ref, x1_ref, w1t_ref, w1b_ref, b1_ref, w2_ref, b2_ref,
             w3_ref, b3_ref, w4_ref, b4_ref, w5_ref, b5_ref, o_ref):
        z = (jnp.dot(x0_ref[...], w1t_ref[...], preferred_element_type=F32)
             + jnp.dot(x1_ref[...], w1b_ref[...], preferred_element_type=F32)
             + b1_ref[...])
        z = jnp.maximum(z, 0.0)
        z = jnp.maximum(jnp.dot(z, w2_ref[...], preferred_element_type=F32)
                        + b2_ref[...], 0.0)
        z = jnp.maximum(jnp.dot(z, w3_ref[...], preferred_element_type=F32)
                        + b3_ref[...], 0.0)
        z = jnp.maximum(jnp.dot(z, w4_ref[...], preferred_element_type=F32)
                        + b4_ref[...], 0.0)
        o_ref[...] = (jnp.dot(z, w5_ref[...], preferred_element_type=F32)
                      + b5_ref[...])

    specs = [
        pl.BlockSpec((n, d), lambda i: (0, 0)),
        pl.BlockSpec((n, d), lambda i: (0, 0)),
        pl.BlockSpec((d, 128), lambda i: (0, 0)),
        pl.BlockSpec((d, 128), lambda i: (1, 0)),
        pl.BlockSpec((1, 128), lambda i: (0, 0)),
        pl.BlockSpec((128, 64), lambda i: (0, 0)),
        pl.BlockSpec((1, 64), lambda i: (0, 0)),
        pl.BlockSpec((64, 32), lambda i: (0, 0)),
        pl.BlockSpec((1, 32), lambda i: (0, 0)),
        pl.BlockSpec((32, 8), lambda i: (0, 0)),
        pl.BlockSpec((1, 8), lambda i: (0, 0)),
        pl.BlockSpec((8, 1), lambda i: (0, 0)),
        pl.BlockSpec((1, 1), lambda i: (0, 0)),
    ]
    return pl.pallas_call(
        body,
        grid=(1,),
        in_specs=specs,
        out_specs=pl.BlockSpec((n, 1), lambda i: (0, 0)),
        out_shape=jax.ShapeDtypeStruct((n, 1), F32),
    )(x0, x1, W1, W1, b1.reshape(1, -1), W2, b2.reshape(1, -1),
      W3, b3.reshape(1, -1), W4, b4.reshape(1, -1), W5, b5.reshape(1, -1))


def _branch_layer1(feats, src_nodes, s2s, s2d, dif, w_agg1):
    idx = jnp.concatenate([jnp.take(src_nodes, s2s, axis=0),
                           jnp.take(src_nodes, s2d, axis=0)])
    g = _sc_gather(feats, idx, chunk=64)
    return _agg_layer(dif, g, w_agg1, relu=True, bm=512, bk=2048,
                      out_dtype=F32)


def _branch_layer2(h1, s2s, s2d, dif, w_agg2):
    idx = jnp.concatenate([s2s, s2d])
    g = _sc_gather(h1, idx, chunk=32)
    return _agg_layer(dif, g, w_agg2, relu=False, bm=1024, bk=2048,
                      out_dtype=F32)


def kernel(feats, src_nodes0, dstsrc2src0_1, dstsrc2src0_2, dstsrc2dst0_1,
           dstsrc2dst0_2, dif_mat0_1, dif_mat0_2, src_nodes1, dstsrc2src1_1,
           dstsrc2src1_2, dstsrc2dst1_1, dstsrc2dst1_2, dif_mat1_1,
           dif_mat1_2, w_agg1, w_agg2, W1, b1, W2, b2, W3, b3, W4, b4, W5,
           b5):
    h1_0 = _branch_layer1(feats, src_nodes0, dstsrc2src0_2, dstsrc2dst0_2,
                          dif_mat0_2, w_agg1)
    h1_1 = _branch_layer1(feats, src_nodes1, dstsrc2src1_2, dstsrc2dst1_2,
                          dif_mat1_2, w_agg1)
    x = _branch_layer2(h1_0, dstsrc2src0_1, dstsrc2dst0_1, dif_mat0_1, w_agg2)
    y = _branch_layer2(h1_1, dstsrc2src1_1, dstsrc2dst1_1, dif_mat1_1, w_agg2)
    return _mlp_head(x, y, W1, b1, W2, b2, W3, b3, W4, b4, W5, b5)


# speedup vs baseline: 1.2029x; 1.0705x over previous
import functools

import jax
import jax.numpy as jnp
from jax import lax
from jax.experimental import pallas as pl
from jax.experimental.pallas import tpu as pltpu
from jax.experimental.pallas import tpu_sc as plsc

F32 = jnp.float32
BF16 = jnp.bfloat16

_NC, _NS = 2, 16
_NW = _NC * _NS


def _sc_gather(table, idx, chunk):
    v, d = table.shape
    b = idx.shape[0]
    b_per_w = b // _NW
    assert b % _NW == 0 and b_per_w % chunk == 0 and chunk % 8 == 0
    n_chunks = b_per_w // chunk
    mesh = plsc.VectorSubcoreMesh(core_axis_name="c", subcore_axis_name="s")

    @functools.partial(
        pl.kernel,
        out_type=jax.ShapeDtypeStruct((b, d), table.dtype),
        mesh=mesh,
        scratch_types=[
            pltpu.VMEM((3, chunk), jnp.int32),
            pltpu.VMEM((3 * chunk, d), table.dtype),
            pltpu.SemaphoreType.DMA((3,)),
            pltpu.SemaphoreType.DMA((3,)),
        ],
    )
    def gather_kernel(table_hbm, idx_hbm, out_hbm, idx_v, rows_v, gsem, wsem):
        wid = lax.axis_index("s") * _NC + lax.axis_index("c")
        base = wid * b_per_w
        gathers = [None, None, None]
        writes = [None, None, None]
        for c in range(n_chunks):
            j = c % 3
            if writes[j] is not None:
                writes[j].wait()
            off = base + c * chunk
            pltpu.sync_copy(idx_hbm.at[pl.ds(off, chunk)], idx_v.at[j])
            gathers[j] = pltpu.async_copy(
                table_hbm.at[idx_v.at[j]], rows_v.at[pl.ds(j * chunk, chunk)],
                gsem.at[j])
            if c >= 1:
                pj = (c - 1) % 3
                gathers[pj].wait()
                poff = base + (c - 1) * chunk
                writes[pj] = pltpu.async_copy(
                    rows_v.at[pl.ds(pj * chunk, chunk)],
                    out_hbm.at[pl.ds(poff, chunk)], wsem.at[pj])
        lj = (n_chunks - 1) % 3
        gathers[lj].wait()
        writes[lj] = pltpu.async_copy(
            rows_v.at[pl.ds(lj * chunk, chunk)],
            out_hbm.at[pl.ds(base + (n_chunks - 1) * chunk, chunk)],
            wsem.at[lj])
        for w in writes:
            if w is not None:
                w.wait()

    return gather_kernel(table, idx)


def _agg_layer(dif, gathered, w, relu, bm, out_dtype):
    m_total, s_total = dif.shape
    d = gathered.shape[1]
    grid = (m_total // bm,)
    dst_block0 = s_total // bm

    def body(dif_ref, src_ref, dst_ref, wt_ref, wb_ref, o_ref, srcb_ref):
        m = pl.program_id(0)

        @pl.when(m == 0)
        def _():
            srcb_ref[...] = src_ref[...].astype(BF16)

        agg = jnp.dot(dif_ref[...].astype(BF16), srcb_ref[...],
                      preferred_element_type=F32)
        o = (jnp.dot(agg.astype(BF16), wt_ref[...].astype(BF16),
                     preferred_element_type=F32)
             + jnp.dot(dst_ref[...].astype(BF16), wb_ref[...].astype(BF16),
                       preferred_element_type=F32))
        if relu:
            o = jnp.maximum(o, 0.0)
        o_ref[...] = o.astype(out_dtype)

    return pl.pallas_call(
        body,
        grid=grid,
        in_specs=[
            pl.BlockSpec((bm, s_total), lambda m: (m, 0)),
            pl.BlockSpec((s_total, d), lambda m: (0, 0)),
            pl.BlockSpec((bm, d), lambda m: (dst_block0 + m, 0)),
            pl.BlockSpec((d, d), lambda m: (0, 0)),
            pl.BlockSpec((d, d), lambda m: (1, 0)),
        ],
        out_specs=pl.BlockSpec((bm, d), lambda m: (m, 0)),
        out_shape=jax.ShapeDtypeStruct((m_total, d), out_dtype),
        scratch_shapes=[pltpu.VMEM((s_total, d), BF16)],
        compiler_params=pltpu.CompilerParams(
            dimension_semantics=("arbitrary",)),
    )(dif, gathered, gathered, w, w)


def _mlp_head(x0, x1, W1, b1, W2, b2, W3, b3, W4, b4, W5, b5):
    n = x0.shape[0]
    d = x0.shape[1]

    def body(x0_ref, x1_ref, w1t_ref, w1b_ref, b1_ref, w2_ref, b2_ref,
             w3_ref, b3_ref, w4_ref, b4_ref, w5_ref, b5_ref, o_ref):
        z = (jnp.dot(x0_ref[...], w1t_ref[...], preferred_element_type=F32)
             + jnp.dot(x1_ref[...], w1b_ref[...], preferred_element_type=F32)
             + b1_ref[...])
        z = jnp.maximum(z, 0.0)
        z = jnp.maximum(jnp.dot(z, w2_ref[...], preferred_element_type=F32)
                        + b2_ref[...], 0.0)
        z = jnp.maximum(jnp.dot(z, w3_ref[...], preferred_element_type=F32)
                        + b3_ref[...], 0.0)
        z = jnp.maximum(jnp.dot(z, w4_ref[...], preferred_element_type=F32)
                        + b4_ref[...], 0.0)
        o_ref[...] = (jnp.dot(z, w5_ref[...], preferred_element_type=F32)
                      + b5_ref[...])

    specs = [
        pl.BlockSpec((n, d), lambda i: (0, 0)),
        pl.BlockSpec((n, d), lambda i: (0, 0)),
        pl.BlockSpec((d, 128), lambda i: (0, 0)),
        pl.BlockSpec((d, 128), lambda i: (1, 0)),
        pl.BlockSpec((1, 128), lambda i: (0, 0)),
        pl.BlockSpec((128, 64), lambda i: (0, 0)),
        pl.BlockSpec((1, 64), lambda i: (0, 0)),
        pl.BlockSpec((64, 32), lambda i: (0, 0)),
        pl.BlockSpec((1, 32), lambda i: (0, 0)),
        pl.BlockSpec((32, 8), lambda i: (0, 0)),
        pl.BlockSpec((1, 8), lambda i: (0, 0)),
        pl.BlockSpec((8, 1), lambda i: (0, 0)),
        pl.BlockSpec((1, 1), lambda i: (0, 0)),
    ]
    return pl.pallas_call(
        body,
        grid=(1,),
        in_specs=specs,
        out_specs=pl.BlockSpec((n, 1), lambda i: (0, 0)),
        out_shape=jax.ShapeDtypeStruct((n, 1), F32),
    )(x0, x1, W1, W1, b1.reshape(1, -1), W2, b2.reshape(1, -1),
      W3, b3.reshape(1, -1), W4, b4.reshape(1, -1), W5, b5.reshape(1, -1))


def _branch_layer1(feats, src_nodes, s2s, s2d, dif, w_agg1):
    idx = jnp.concatenate([jnp.take(src_nodes, s2s, axis=0),
                           jnp.take(src_nodes, s2d, axis=0)])
    g = _sc_gather(feats, idx, chunk=64)
    return _agg_layer(dif, g, w_agg1, relu=True, bm=256, out_dtype=F32)


def _branch_layer2(h1, s2s, s2d, dif, w_agg2):
    idx = jnp.concatenate([s2s, s2d])
    g = _sc_gather(h1, idx, chunk=32)
    return _agg_layer(dif, g, w_agg2, relu=False, bm=512, out_dtype=F32)


def kernel(feats, src_nodes0, dstsrc2src0_1, dstsrc2src0_2, dstsrc2dst0_1,
           dstsrc2dst0_2, dif_mat0_1, dif_mat0_2, src_nodes1, dstsrc2src1_1,
           dstsrc2src1_2, dstsrc2dst1_1, dstsrc2dst1_2, dif_mat1_1,
           dif_mat1_2, w_agg1, w_agg2, W1, b1, W2, b2, W3, b3, W4, b4, W5,
           b5):
    h1_0 = _branch_layer1(feats, src_nodes0, dstsrc2src0_2, dstsrc2dst0_2,
                          dif_mat0_2, w_agg1)
    h1_1 = _branch_layer1(feats, src_nodes1, dstsrc2src1_2, dstsrc2dst1_2,
                          dif_mat1_2, w_agg1)
    x = _branch_layer2(h1_0, dstsrc2src0_1, dstsrc2dst0_1, dif_mat0_1, w_agg2)
    y = _branch_layer2(h1_1, dstsrc2src1_1, dstsrc2dst1_1, dif_mat1_1, w_agg2)
    return _mlp_head(x, y, W1, b1, W2, b2, W3, b3, W4, b4, W5, b5)


# speedup vs baseline: 1.3034x; 1.0836x over previous
import dataclasses
import functools

import jax
import jax.numpy as jnp
from jax import lax
from jax.experimental import pallas as pl
from jax.experimental.pallas import tpu as pltpu
from jax.experimental.pallas import tpu_sc as plsc

F32 = jnp.float32
BF16 = jnp.bfloat16

_NC, _NS = 2, 16
_NW = _NC * _NS


def _sc_gather(table, idx, chunk, nodes=None):
    v, d = table.shape
    b = idx.shape[0]
    b_per_w = b // _NW
    assert b % _NW == 0 and b_per_w % chunk == 0 and chunk % 16 == 0
    n_chunks = b_per_w // chunk
    n_nodes = 0 if nodes is None else nodes.shape[0]
    mesh = plsc.VectorSubcoreMesh(core_axis_name="c", subcore_axis_name="s")

    scratch = [
        pltpu.VMEM((3, chunk), jnp.int32),
        pltpu.VMEM((3 * chunk, d), table.dtype),
        pltpu.SemaphoreType.DMA((3,)),
        pltpu.SemaphoreType.DMA((3,)),
    ]
    if nodes is not None:
        scratch.append(pltpu.VMEM((n_nodes,), jnp.int32))

    def gather_body(table_hbm, idx_hbm, *rest):
        if nodes is not None:
            nodes_hbm, out_hbm, idx_v, rows_v, gsem, wsem, nodes_v = rest
            pltpu.sync_copy(nodes_hbm, nodes_v)
        else:
            out_hbm, idx_v, rows_v, gsem, wsem = rest
        wid = lax.axis_index("s") * _NC + lax.axis_index("c")
        base = wid * b_per_w
        gathers = [None, None, None]
        writes = [None, None, None]
        for c in range(n_chunks):
            j = c % 3
            if writes[j] is not None:
                writes[j].wait()
            off = base + c * chunk
            pltpu.sync_copy(idx_hbm.at[pl.ds(off, chunk)], idx_v.at[j])
            if nodes is not None:
                @pl.loop(0, chunk, step=16)
                def _(t, _j=j):
                    raw = idx_v[_j, pl.ds(t, 16)]
                    idx_v[_j, pl.ds(t, 16)] = plsc.load_gather(nodes_v, [raw])
            gathers[j] = pltpu.async_copy(
                table_hbm.at[idx_v.at[j]], rows_v.at[pl.ds(j * chunk, chunk)],
                gsem.at[j])
            if c >= 1:
                pj = (c - 1) % 3
                gathers[pj].wait()
                poff = base + (c - 1) * chunk
                writes[pj] = pltpu.async_copy(
                    rows_v.at[pl.ds(pj * chunk, chunk)],
                    out_hbm.at[pl.ds(poff, chunk)], wsem.at[pj])
        lj = (n_chunks - 1) % 3
        gathers[lj].wait()
        writes[lj] = pltpu.async_copy(
            rows_v.at[pl.ds(lj * chunk, chunk)],
            out_hbm.at[pl.ds(base + (n_chunks - 1) * chunk, chunk)],
            wsem.at[lj])
        for w in writes:
            if w is not None:
                w.wait()

    cp = pltpu.CompilerParams()
    if "needs_layout_passes" in pltpu.CompilerParams.__dataclass_fields__:
        cp = dataclasses.replace(cp, needs_layout_passes=False)
    kern = functools.partial(
        pl.kernel,
        out_type=jax.ShapeDtypeStruct((b, d), table.dtype),
        mesh=mesh,
        scratch_types=scratch,
        compiler_params=cp,
    )(gather_body)
    if nodes is not None:
        return kern(table, idx, nodes)
    return kern(table, idx)


def _agg_layer(dif, gathered, w, relu, bm, out_dtype):
    m_total, s_total = dif.shape
    d = gathered.shape[1]
    grid = (m_total // bm,)
    dst_block0 = s_total // bm

    def body(dif_ref, src_ref, dst_ref, wt_ref, wb_ref, o_ref, srcb_ref):
        m = pl.program_id(0)

        @pl.when(m == 0)
        def _():
            srcb_ref[...] = src_ref[...].astype(BF16)

        agg = jnp.dot(dif_ref[...].astype(BF16), srcb_ref[...],
                      preferred_element_type=F32)
        o = (jnp.dot(agg.astype(BF16), wt_ref[...].astype(BF16),
                     preferred_element_type=F32)
             + jnp.dot(dst_ref[...].astype(BF16), wb_ref[...].astype(BF16),
                       preferred_element_type=F32))
        if relu:
            o = jnp.maximum(o, 0.0)
        o_ref[...] = o.astype(out_dtype)

    return pl.pallas_call(
        body,
        grid=grid,
        in_specs=[
            pl.BlockSpec((bm, s_total), lambda m: (m, 0)),
            pl.BlockSpec((s_total, d), lambda m: (0, 0)),
            pl.BlockSpec((bm, d), lambda m: (dst_block0 + m, 0)),
            pl.BlockSpec((d, d), lambda m: (0, 0)),
            pl.BlockSpec((d, d), lambda m: (1, 0)),
        ],
        out_specs=pl.BlockSpec((bm, d), lambda m: (m, 0)),
        out_shape=jax.ShapeDtypeStruct((m_total, d), out_dtype),
        scratch_shapes=[pltpu.VMEM((s_total, d), BF16)],
        compiler_params=pltpu.CompilerParams(
            dimension_semantics=("arbitrary",)),
    )(dif, gathered, gathered, w, w)


def _mlp_head(x0, x1, W1, b1, W2, b2, W3, b3, W4, b4, W5, b5):
    n = x0.shape[0]
    d = x0.shape[1]

    def body(x0_ref, x1_ref, w1t_ref, w1b_ref, b1_ref, w2_ref, b2_ref,
             w3_ref, b3_ref, w4_ref, b4_ref, w5_ref, b5_ref, o_ref):
        z = (jnp.dot(x0_ref[...], w1t_ref[...], preferred_element_type=F32)
             + jnp.dot(x1_ref[...], w1b_ref[...], preferred_element_type=F32)
             + b1_ref[...])
        z = jnp.maximum(z, 0.0)
        z = jnp.maximum(jnp.dot(z, w2_ref[...], preferred_element_type=F32)
                        + b2_ref[...], 0.0)
        z = jnp.maximum(jnp.dot(z, w3_ref[...], preferred_element_type=F32)
                        + b3_ref[...], 0.0)
        z = jnp.maximum(jnp.dot(z, w4_ref[...], preferred_element_type=F32)
                        + b4_ref[...], 0.0)
        o_ref[...] = (jnp.dot(z, w5_ref[...], preferred_element_type=F32)
                      + b5_ref[...])

    specs = [
        pl.BlockSpec((n, d), lambda i: (0, 0)),
        pl.BlockSpec((n, d), lambda i: (0, 0)),
        pl.BlockSpec((d, 128), lambda i: (0, 0)),
        pl.BlockSpec((d, 128), lambda i: (1, 0)),
        pl.BlockSpec((1, 128), lambda i: (0, 0)),
        pl.BlockSpec((128, 64), lambda i: (0, 0)),
        pl.BlockSpec((1, 64), lambda i: (0, 0)),
        pl.BlockSpec((64, 32), lambda i: (0, 0)),
        pl.BlockSpec((1, 32), lambda i: (0, 0)),
        pl.BlockSpec((32, 8), lambda i: (0, 0)),
        pl.BlockSpec((1, 8), lambda i: (0, 0)),
        pl.BlockSpec((8, 1), lambda i: (0, 0)),
        pl.BlockSpec((1, 1), lambda i: (0, 0)),
    ]
    return pl.pallas_call(
        body,
        grid=(1,),
        in_specs=specs,
        out_specs=pl.BlockSpec((n, 1), lambda i: (0, 0)),
        out_shape=jax.ShapeDtypeStruct((n, 1), F32),
    )(x0, x1, W1, W1, b1.reshape(1, -1), W2, b2.reshape(1, -1),
      W3, b3.reshape(1, -1), W4, b4.reshape(1, -1), W5, b5.reshape(1, -1))


def _branch_layer1(feats, src_nodes, s2s, s2d, dif, w_agg1):
    idx = jnp.concatenate([s2s, s2d])
    g = _sc_gather(feats, idx, chunk=64, nodes=src_nodes)
    return _agg_layer(dif, g, w_agg1, relu=True, bm=256, out_dtype=F32)


def _branch_layer2(h1, s2s, s2d, dif, w_agg2):
    idx = jnp.concatenate([s2s, s2d])
    g = _sc_gather(h1, idx, chunk=32)
    return _agg_layer(dif, g, w_agg2, relu=False, bm=512, out_dtype=F32)


def kernel(feats, src_nodes0, dstsrc2src0_1, dstsrc2src0_2, dstsrc2dst0_1,
           dstsrc2dst0_2, dif_mat0_1, dif_mat0_2, src_nodes1, dstsrc2src1_1,
           dstsrc2src1_2, dstsrc2dst1_1, dstsrc2dst1_2, dif_mat1_1,
           dif_mat1_2, w_agg1, w_agg2, W1, b1, W2, b2, W3, b3, W4, b4, W5,
           b5):
    h1_0 = _branch_layer1(feats, src_nodes0, dstsrc2src0_2, dstsrc2dst0_2,
                          dif_mat0_2, w_agg1)
    h1_1 = _branch_layer1(feats, src_nodes1, dstsrc2src1_2, dstsrc2dst1_2,
                          dif_mat1_2, w_agg1)
    x = _branch_layer2(h1_0, dstsrc2src0_1, dstsrc2dst0_1, dif_mat0_1, w_agg2)
    y = _branch_layer2(h1_1, dstsrc2src1_1, dstsrc2dst1_1, dif_mat1_1, w_agg2)
    return _mlp_head(x, y, W1, b1, W2, b2, W3, b3, W4, b4, W5, b5)


# speedup vs baseline: 1.4561x; 1.1171x over previous
import dataclasses
import functools

import jax
import jax.numpy as jnp
from jax import lax
from jax.experimental import pallas as pl
from jax.experimental.pallas import tpu as pltpu
from jax.experimental.pallas import tpu_sc as plsc

F32 = jnp.float32
BF16 = jnp.bfloat16

_NC, _NS = 2, 16
_NW = _NC * _NS


def _sc_gather(table, idx, chunk, nodes=None):
    v, d = table.shape
    b = idx.shape[0]
    b_per_w = b // _NW
    assert b % _NW == 0 and b_per_w % chunk == 0 and chunk % 16 == 0
    n_chunks = b_per_w // chunk
    n_nodes = 0 if nodes is None else nodes.shape[0]
    mesh = plsc.VectorSubcoreMesh(core_axis_name="c", subcore_axis_name="s")

    scratch = [
        pltpu.VMEM((3, chunk), jnp.int32),
        pltpu.VMEM((3 * chunk, d), table.dtype),
        pltpu.SemaphoreType.DMA((3,)),
        pltpu.SemaphoreType.DMA((3,)),
    ]
    if nodes is not None:
        scratch.append(pltpu.VMEM((n_nodes,), jnp.int32))

    def gather_body(table_hbm, idx_hbm, *rest):
        if nodes is not None:
            nodes_hbm, out_hbm, idx_v, rows_v, gsem, wsem, nodes_v = rest
            pltpu.sync_copy(nodes_hbm, nodes_v)
        else:
            out_hbm, idx_v, rows_v, gsem, wsem = rest
        wid = lax.axis_index("s") * _NC + lax.axis_index("c")
        base = wid * b_per_w
        gathers = [None, None, None]
        writes = [None, None, None]
        for c in range(n_chunks):
            j = c % 3
            if writes[j] is not None:
                writes[j].wait()
            off = base + c * chunk
            pltpu.sync_copy(idx_hbm.at[pl.ds(off, chunk)], idx_v.at[j])
            if nodes is not None:
                @pl.loop(0, chunk, step=16)
                def _(t, _j=j):
                    raw = idx_v[_j, pl.ds(t, 16)]
                    idx_v[_j, pl.ds(t, 16)] = plsc.load_gather(nodes_v, [raw])
            gathers[j] = pltpu.async_copy(
                table_hbm.at[idx_v.at[j]], rows_v.at[pl.ds(j * chunk, chunk)],
                gsem.at[j])
            if c >= 1:
                pj = (c - 1) % 3
                gathers[pj].wait()
                poff = base + (c - 1) * chunk
                writes[pj] = pltpu.async_copy(
                    rows_v.at[pl.ds(pj * chunk, chunk)],
                    out_hbm.at[pl.ds(poff, chunk)], wsem.at[pj])
        lj = (n_chunks - 1) % 3
        gathers[lj].wait()
        writes[lj] = pltpu.async_copy(
            rows_v.at[pl.ds(lj * chunk, chunk)],
            out_hbm.at[pl.ds(base + (n_chunks - 1) * chunk, chunk)],
            wsem.at[lj])
        for w in writes:
            if w is not None:
                w.wait()

    cp = pltpu.CompilerParams()
    if "needs_layout_passes" in pltpu.CompilerParams.__dataclass_fields__:
        cp = dataclasses.replace(cp, needs_layout_passes=False)
    kern = functools.partial(
        pl.kernel,
        out_type=jax.ShapeDtypeStruct((b, d), table.dtype),
        mesh=mesh,
        scratch_types=scratch,
        compiler_params=cp,
    )(gather_body)
    if nodes is not None:
        return kern(table, idx, nodes)
    return kern(table, idx)


def _agg_layer(dif, gathered, w, relu, bm, out_packed):
    m_total, s_total = dif.shape
    d = 2 * gathered.shape[1]
    hd = d // 2
    grid = (m_total // bm,)
    dst_block0 = s_total // bm

    def unpack(w_i32):
        wu = jax.lax.bitcast_convert_type(w_i32, jnp.uint32)
        lo = jax.lax.bitcast_convert_type(wu << 16, F32)
        hi = jax.lax.bitcast_convert_type(wu & jnp.uint32(0xFFFF0000), F32)
        return lo.astype(BF16), hi.astype(BF16)

    def body(dif_ref, src_ref, dst_ref, wt_ref, wb_ref, o_ref, srcb_ref):
        m = pl.program_id(0)

        @pl.when(m == 0)
        def _():
            lo, hi = unpack(src_ref[...])
            srcb_ref[:, :hd] = lo
            srcb_ref[:, hd:] = hi

        agg = jnp.dot(dif_ref[...].astype(BF16), srcb_ref[...],
                      preferred_element_type=F32)
        dlo, dhi = unpack(dst_ref[...])
        o = (jnp.dot(agg.astype(BF16), wt_ref[...].astype(BF16),
                     preferred_element_type=F32)
             + jnp.dot(dlo, wb_ref[pl.ds(0, hd), :].astype(BF16),
                       preferred_element_type=F32)
             + jnp.dot(dhi, wb_ref[pl.ds(hd, hd), :].astype(BF16),
                       preferred_element_type=F32))
        if relu:
            o = jnp.maximum(o, 0.0)
        if out_packed:
            lo = jax.lax.bitcast_convert_type(o[:, :hd], jnp.uint32)
            hi = jax.lax.bitcast_convert_type(o[:, hd:], jnp.uint32)
            packed = (((hi + 0x8000) & jnp.uint32(0xFFFF0000))
                      | ((lo + 0x8000) >> 16))
            o_ref[...] = jax.lax.bitcast_convert_type(packed, jnp.int32)
        else:
            o_ref[...] = o

    out_cols = hd if out_packed else d
    out_dtype = jnp.int32 if out_packed else F32
    return pl.pallas_call(
        body,
        grid=grid,
        in_specs=[
            pl.BlockSpec((bm, s_total), lambda m: (m, 0)),
            pl.BlockSpec((s_total, hd), lambda m: (0, 0)),
            pl.BlockSpec((bm, hd), lambda m: (dst_block0 + m, 0)),
            pl.BlockSpec((d, d), lambda m: (0, 0)),
            pl.BlockSpec((d, d), lambda m: (1, 0)),
        ],
        out_specs=pl.BlockSpec((bm, out_cols), lambda m: (m, 0)),
        out_shape=jax.ShapeDtypeStruct((m_total, out_cols), out_dtype),
        scratch_shapes=[pltpu.VMEM((s_total, d), BF16)],
        compiler_params=pltpu.CompilerParams(
            dimension_semantics=("arbitrary",)),
    )(dif, gathered, gathered, w, w)


def _mlp_head(x0, x1, W1, b1, W2, b2, W3, b3, W4, b4, W5, b5):
    n = x0.shape[0]
    d = x0.shape[1]

    def body(x0_ref, x1_ref, w1t_ref, w1b_ref, b1_ref, w2_ref, b2_ref,
             w3_ref, b3_ref, w4_ref, b4_ref, w5_ref, b5_ref, o_ref):
        z = (jnp.dot(x0_ref[...], w1t_ref[...], preferred_element_type=F32)
             + jnp.dot(x1_ref[...], w1b_ref[...], preferred_element_type=F32)
             + b1_ref[...])
        z = jnp.maximum(z, 0.0)
        z = jnp.maximum(jnp.dot(z, w2_ref[...], preferred_element_type=F32)
                        + b2_ref[...], 0.0)
        z = jnp.maximum(jnp.dot(z, w3_ref[...], preferred_element_type=F32)
                        + b3_ref[...], 0.0)
        z = jnp.maximum(jnp.dot(z, w4_ref[...], preferred_element_type=F32)
                        + b4_ref[...], 0.0)
        o_ref[...] = (jnp.dot(z, w5_ref[...], preferred_element_type=F32)
                      + b5_ref[...])

    specs = [
        pl.BlockSpec((n, d), lambda i: (0, 0)),
        pl.BlockSpec((n, d), lambda i: (0, 0)),
        pl.BlockSpec((d, 128), lambda i: (0, 0)),
        pl.BlockSpec((d, 128), lambda i: (1, 0)),
        pl.BlockSpec((1, 128), lambda i: (0, 0)),
        pl.BlockSpec((128, 64), lambda i: (0, 0)),
        pl.BlockSpec((1, 64), lambda i: (0, 0)),
        pl.BlockSpec((64, 32), lambda i: (0, 0)),
        pl.BlockSpec((1, 32), lambda i: (0, 0)),
        pl.BlockSpec((32, 8), lambda i: (0, 0)),
        pl.BlockSpec((1, 8), lambda i: (0, 0)),
        pl.BlockSpec((8, 1), lambda i: (0, 0)),
        pl.BlockSpec((1, 1), lambda i: (0, 0)),
    ]
    return pl.pallas_call(
        body,
        grid=(1,),
        in_specs=specs,
        out_specs=pl.BlockSpec((n, 1), lambda i: (0, 0)),
        out_shape=jax.ShapeDtypeStruct((n, 1), F32),
    )(x0, x1, W1, W1, b1.reshape(1, -1), W2, b2.reshape(1, -1),
      W3, b3.reshape(1, -1), W4, b4.reshape(1, -1), W5, b5.reshape(1, -1))


def _pack_halves(x):
    h = x.shape[1] // 2
    lo = jax.lax.bitcast_convert_type(x[:, :h], jnp.uint32)
    hi = jax.lax.bitcast_convert_type(x[:, h:], jnp.uint32)
    packed = (((hi + 0x8000) & jnp.uint32(0xFFFF0000)) | ((lo + 0x8000) >> 16))
    return jax.lax.bitcast_convert_type(packed, jnp.int32)


def _branch_layer1(feats_p, src_nodes, s2s, s2d, dif, w_agg1):
    idx = jnp.concatenate([s2s, s2d])
    g = _sc_gather(feats_p, idx, chunk=128, nodes=src_nodes)
    return _agg_layer(dif, g, w_agg1, relu=True, bm=256, out_packed=True)


def _branch_layer2(h1_p, s2s, s2d, dif, w_agg2):
    idx = jnp.concatenate([s2s, s2d])
    g = _sc_gather(h1_p, idx, chunk=32)
    return _agg_layer(dif, g, w_agg2, relu=False, bm=512, out_packed=False)


def kernel(feats, src_nodes0, dstsrc2src0_1, dstsrc2src0_2, dstsrc2dst0_1,
           dstsrc2dst0_2, dif_mat0_1, dif_mat0_2, src_nodes1, dstsrc2src1_1,
           dstsrc2src1_2, dstsrc2dst1_1, dstsrc2dst1_2, dif_mat1_1,
           dif_mat1_2, w_agg1, w_agg2, W1, b1, W2, b2, W3, b3, W4, b4, W5,
           b5):
    feats_p = _pack_halves(feats)
    h1_0 = _branch_layer1(feats_p, src_nodes0, dstsrc2src0_2, dstsrc2dst0_2,
                          dif_mat0_2, w_agg1)
    h1_1 = _branch_layer1(feats_p, src_nodes1, dstsrc2src1_2, dstsrc2dst1_2,
                          dif_mat1_2, w_agg1)
    x = _branch_layer2(h1_0, dstsrc2src0_1, dstsrc2dst0_1, dif_mat0_1, w_agg2)
    y = _branch_layer2(h1_1, dstsrc2src1_1, dstsrc2dst1_1, dif_mat1_1, w_agg2)
    return _mlp_head(x, y, W1, b1, W2, b2, W3, b3, W4, b4, W5, b5)


# speedup vs baseline: 1.5093x; 1.0366x over previous
import dataclasses
import functools

import jax
import jax.numpy as jnp
from jax import lax
from jax.experimental import pallas as pl
from jax.experimental.pallas import tpu as pltpu
from jax.experimental.pallas import tpu_sc as plsc

F32 = jnp.float32
BF16 = jnp.bfloat16

_NC, _NS = 2, 16
_NW = _NC * _NS


def _sc_gather(table, idx, chunk, nodes=None):
    v, d = table.shape
    b = idx.shape[0]
    b_per_w = b // _NW
    assert b % _NW == 0 and b_per_w % chunk == 0 and chunk % 16 == 0
    n_chunks = b_per_w // chunk
    n_nodes = 0 if nodes is None else nodes.shape[0]
    mesh = plsc.VectorSubcoreMesh(core_axis_name="c", subcore_axis_name="s")

    scratch = [
        pltpu.VMEM((3, chunk), jnp.int32),
        pltpu.VMEM((3 * chunk, d), table.dtype),
        pltpu.SemaphoreType.DMA((3,)),
        pltpu.SemaphoreType.DMA((3,)),
    ]
    if nodes is not None:
        scratch.append(pltpu.VMEM((n_nodes,), jnp.int32))

    def gather_body(table_hbm, idx_hbm, *rest):
        if nodes is not None:
            nodes_hbm, out_hbm, idx_v, rows_v, gsem, wsem, nodes_v = rest
            pltpu.sync_copy(nodes_hbm, nodes_v)
        else:
            out_hbm, idx_v, rows_v, gsem, wsem = rest
        wid = lax.axis_index("s") * _NC + lax.axis_index("c")
        base = wid * b_per_w
        gathers = [None, None, None]
        writes = [None, None, None]
        for c in range(n_chunks):
            j = c % 3
            if writes[j] is not None:
                writes[j].wait()
            off = base + c * chunk
            pltpu.sync_copy(idx_hbm.at[pl.ds(off, chunk)], idx_v.at[j])
            if nodes is not None:
                @pl.loop(0, chunk, step=16)
                def _(t, _j=j):
                    raw = idx_v[_j, pl.ds(t, 16)]
                    idx_v[_j, pl.ds(t, 16)] = plsc.load_gather(nodes_v, [raw])
            gathers[j] = pltpu.async_copy(
                table_hbm.at[idx_v.at[j]], rows_v.at[pl.ds(j * chunk, chunk)],
                gsem.at[j])
            if c >= 1:
                pj = (c - 1) % 3
                gathers[pj].wait()
                poff = base + (c - 1) * chunk
                writes[pj] = pltpu.async_copy(
                    rows_v.at[pl.ds(pj * chunk, chunk)],
                    out_hbm.at[pl.ds(poff, chunk)], wsem.at[pj])
        lj = (n_chunks - 1) % 3
        gathers[lj].wait()
        writes[lj] = pltpu.async_copy(
            rows_v.at[pl.ds(lj * chunk, chunk)],
            out_hbm.at[pl.ds(base + (n_chunks - 1) * chunk, chunk)],
            wsem.at[lj])
        for w in writes:
            if w is not None:
                w.wait()

    cp = pltpu.CompilerParams()
    if "needs_layout_passes" in pltpu.CompilerParams.__dataclass_fields__:
        cp = dataclasses.replace(cp, needs_layout_passes=False)
    kern = functools.partial(
        pl.kernel,
        out_type=jax.ShapeDtypeStruct((b, d), table.dtype),
        mesh=mesh,
        scratch_types=scratch,
        compiler_params=cp,
    )(gather_body)
    if nodes is not None:
        return kern(table, idx, nodes)
    return kern(table, idx)


def _agg_layer(dif, gathered, w, relu, bm, out_packed):
    m_total, s_total = dif.shape
    d = 2 * gathered.shape[1]
    hd = d // 2
    grid = (m_total // bm,)
    dst_block0 = s_total // bm

    def unpack(w_i32):
        wu = jax.lax.bitcast_convert_type(w_i32, jnp.uint32)
        lo = jax.lax.bitcast_convert_type(wu << 16, F32)
        hi = jax.lax.bitcast_convert_type(wu & jnp.uint32(0xFFFF0000), F32)
        return lo.astype(BF16), hi.astype(BF16)

    sh = s_total // 2

    def body(difl_ref, difr_ref, src_ref, dst_ref, wt_ref, wb_ref, o_ref,
             srcb_ref):
        m = pl.program_id(0)

        @pl.when(m == 0)
        def _():
            lo, hi = unpack(src_ref[...])
            srcb_ref[:, :hd] = lo
            srcb_ref[:, hd:] = hi

        agg = (jnp.dot(difl_ref[...].astype(BF16), srcb_ref[pl.ds(0, sh), :],
                       preferred_element_type=F32)
               + jnp.dot(difr_ref[...].astype(BF16),
                         srcb_ref[pl.ds(sh, sh), :],
                         preferred_element_type=F32))
        dlo, dhi = unpack(dst_ref[...])
        o = (jnp.dot(agg.astype(BF16), wt_ref[...].astype(BF16),
                     preferred_element_type=F32)
             + jnp.dot(dlo, wb_ref[pl.ds(0, hd), :].astype(BF16),
                       preferred_element_type=F32)
             + jnp.dot(dhi, wb_ref[pl.ds(hd, hd), :].astype(BF16),
                       preferred_element_type=F32))
        if relu:
            o = jnp.maximum(o, 0.0)
        if out_packed:
            lo = jax.lax.bitcast_convert_type(o[:, :hd], jnp.uint32)
            hi = jax.lax.bitcast_convert_type(o[:, hd:], jnp.uint32)
            packed = (((hi + 0x8000) & jnp.uint32(0xFFFF0000))
                      | ((lo + 0x8000) >> 16))
            o_ref[...] = jax.lax.bitcast_convert_type(packed, jnp.int32)
        else:
            o_ref[...] = o

    out_cols = hd if out_packed else d
    out_dtype = jnp.int32 if out_packed else F32
    return pl.pallas_call(
        body,
        grid=grid,
        in_specs=[
            pl.BlockSpec((bm, sh), lambda m: (m, 0)),
            pl.BlockSpec((bm, sh), lambda m: (m, 1)),
            pl.BlockSpec((s_total, hd), lambda m: (0, 0)),
            pl.BlockSpec((bm, hd), lambda m: (dst_block0 + m, 0)),
            pl.BlockSpec((d, d), lambda m: (0, 0)),
            pl.BlockSpec((d, d), lambda m: (1, 0)),
        ],
        out_specs=pl.BlockSpec((bm, out_cols), lambda m: (m, 0)),
        out_shape=jax.ShapeDtypeStruct((m_total, out_cols), out_dtype),
        scratch_shapes=[pltpu.VMEM((s_total, d), BF16)],
        compiler_params=pltpu.CompilerParams(
            dimension_semantics=("arbitrary",)),
    )(dif, dif, gathered, gathered, w, w)


def _mlp_head(x0, x1, W1, b1, W2, b2, W3, b3, W4, b4, W5, b5):
    n = x0.shape[0]
    d = x0.shape[1]

    def body(x0_ref, x1_ref, w1t_ref, w1b_ref, b1_ref, w2_ref, b2_ref,
             w3_ref, b3_ref, w4_ref, b4_ref, w5_ref, b5_ref, o_ref):
        z = (jnp.dot(x0_ref[...], w1t_ref[...], preferred_element_type=F32)
             + jnp.dot(x1_ref[...], w1b_ref[...], preferred_element_type=F32)
             + b1_ref[...])
        z = jnp.maximum(z, 0.0)
        z = jnp.maximum(jnp.dot(z, w2_ref[...], preferred_element_type=F32)
                        + b2_ref[...], 0.0)
        z = jnp.maximum(jnp.dot(z, w3_ref[...], preferred_element_type=F32)
                        + b3_ref[...], 0.0)
        z = jnp.maximum(jnp.dot(z, w4_ref[...], preferred_element_type=F32)
                        + b4_ref[...], 0.0)
        o_ref[...] = (jnp.dot(z, w5_ref[...], preferred_element_type=F32)
                      + b5_ref[...])

    specs = [
        pl.BlockSpec((n, d), lambda i: (0, 0)),
        pl.BlockSpec((n, d), lambda i: (0, 0)),
        pl.BlockSpec((d, 128), lambda i: (0, 0)),
        pl.BlockSpec((d, 128), lambda i: (1, 0)),
        pl.BlockSpec((1, 128), lambda i: (0, 0)),
        pl.BlockSpec((128, 64), lambda i: (0, 0)),
        pl.BlockSpec((1, 64), lambda i: (0, 0)),
        pl.BlockSpec((64, 32), lambda i: (0, 0)),
        pl.BlockSpec((1, 32), lambda i: (0, 0)),
        pl.BlockSpec((32, 8), lambda i: (0, 0)),
        pl.BlockSpec((1, 8), lambda i: (0, 0)),
        pl.BlockSpec((8, 1), lambda i: (0, 0)),
        pl.BlockSpec((1, 1), lambda i: (0, 0)),
    ]
    return pl.pallas_call(
        body,
        grid=(1,),
        in_specs=specs,
        out_specs=pl.BlockSpec((n, 1), lambda i: (0, 0)),
        out_shape=jax.ShapeDtypeStruct((n, 1), F32),
    )(x0, x1, W1, W1, b1.reshape(1, -1), W2, b2.reshape(1, -1),
      W3, b3.reshape(1, -1), W4, b4.reshape(1, -1), W5, b5.reshape(1, -1))


def _pack_halves(x):
    h = x.shape[1] // 2
    lo = jax.lax.bitcast_convert_type(x[:, :h], jnp.uint32)
    hi = jax.lax.bitcast_convert_type(x[:, h:], jnp.uint32)
    packed = (((hi + 0x8000) & jnp.uint32(0xFFFF0000)) | ((lo + 0x8000) >> 16))
    return jax.lax.bitcast_convert_type(packed, jnp.int32)


def _branch_layer1(feats_p, src_nodes, s2s, s2d, dif, w_agg1):
    idx = jnp.concatenate([s2s, s2d])
    g = _sc_gather(feats_p, idx, chunk=128, nodes=src_nodes)
    return _agg_layer(dif, g, w_agg1, relu=True, bm=256, out_packed=True)


def _branch_layer2(h1_p, s2s, s2d, dif, w_agg2):
    idx = jnp.concatenate([s2s, s2d])
    g = _sc_gather(h1_p, idx, chunk=32)
    return _agg_layer(dif, g, w_agg2, relu=False, bm=256, out_packed=False)


def kernel(feats, src_nodes0, dstsrc2src0_1, dstsrc2src0_2, dstsrc2dst0_1,
           dstsrc2dst0_2, dif_mat0_1, dif_mat0_2, src_nodes1, dstsrc2src1_1,
           dstsrc2src1_2, dstsrc2dst1_1, dstsrc2dst1_2, dif_mat1_1,
           dif_mat1_2, w_agg1, w_agg2, W1, b1, W2, b2, W3, b3, W4, b4, W5,
           b5):
    feats_p = _pack_halves(feats)
    h1_0 = _branch_layer1(feats_p, src_nodes0, dstsrc2src0_2, dstsrc2dst0_2,
                          dif_mat0_2, w_agg1)
    h1_1 = _branch_layer1(feats_p, src_nodes1, dstsrc2src1_2, dstsrc2dst1_2,
                          dif_mat1_2, w_agg1)
    x = _branch_layer2(h1_0, dstsrc2src0_1, dstsrc2dst0_1, dif_mat0_1, w_agg2)
    y = _branch_layer2(h1_1, dstsrc2src1_1, dstsrc2dst1_1, dif_mat1_1, w_agg2)
    return _mlp_head(x, y, W1, b1, W2, b2, W3, b3, W4, b4, W5, b5)


# speedup vs baseline: 1.5250x; 1.0104x over previous
import dataclasses
import functools

import jax
import jax.numpy as jnp
from jax import lax
from jax.experimental import pallas as pl
from jax.experimental.pallas import tpu as pltpu
from jax.experimental.pallas import tpu_sc as plsc

F32 = jnp.float32
BF16 = jnp.bfloat16

_NC, _NS = 2, 16
_NW = _NC * _NS


def _sc_gather(table, idx, chunk, nodes=None):
    v, d = table.shape
    b = idx.shape[0]
    b_per_w = b // _NW
    assert b % _NW == 0 and b_per_w % chunk == 0 and chunk % 16 == 0
    n_chunks = b_per_w // chunk
    n_nodes = 0 if nodes is None else nodes.shape[0]
    mesh = plsc.VectorSubcoreMesh(core_axis_name="c", subcore_axis_name="s")

    scratch = [
        pltpu.VMEM((3, chunk), jnp.int32),
        pltpu.VMEM((3 * chunk, d), table.dtype),
        pltpu.SemaphoreType.DMA((3,)),
        pltpu.SemaphoreType.DMA((3,)),
    ]
    if nodes is not None:
        scratch.append(pltpu.VMEM((n_nodes,), jnp.int32))

    def gather_body(table_hbm, idx_hbm, *rest):
        if nodes is not None:
            nodes_hbm, out_hbm, idx_v, rows_v, gsem, wsem, nodes_v = rest
            pltpu.sync_copy(nodes_hbm, nodes_v)
        else:
            out_hbm, idx_v, rows_v, gsem, wsem = rest
        wid = lax.axis_index("s") * _NC + lax.axis_index("c")
        base = wid * b_per_w
        gathers = [None, None, None]
        writes = [None, None, None]
        for c in range(n_chunks):
            j = c % 3
            if writes[j] is not None:
                writes[j].wait()
            off = base + c * chunk
            pltpu.sync_copy(idx_hbm.at[pl.ds(off, chunk)], idx_v.at[j])
            if nodes is not None:
                @pl.loop(0, chunk, step=16)
                def _(t, _j=j):
                    raw = idx_v[_j, pl.ds(t, 16)]
                    idx_v[_j, pl.ds(t, 16)] = plsc.load_gather(nodes_v, [raw])
            gathers[j] = pltpu.async_copy(
                table_hbm.at[idx_v.at[j]], rows_v.at[pl.ds(j * chunk, chunk)],
                gsem.at[j])
            if c >= 1:
                pj = (c - 1) % 3
                gathers[pj].wait()
                poff = base + (c - 1) * chunk
                writes[pj] = pltpu.async_copy(
                    rows_v.at[pl.ds(pj * chunk, chunk)],
                    out_hbm.at[pl.ds(poff, chunk)], wsem.at[pj])
        lj = (n_chunks - 1) % 3
        gathers[lj].wait()
        writes[lj] = pltpu.async_copy(
            rows_v.at[pl.ds(lj * chunk, chunk)],
            out_hbm.at[pl.ds(base + (n_chunks - 1) * chunk, chunk)],
            wsem.at[lj])
        for w in writes:
            if w is not None:
                w.wait()

    cp = pltpu.CompilerParams()
    if "needs_layout_passes" in pltpu.CompilerParams.__dataclass_fields__:
        cp = dataclasses.replace(cp, needs_layout_passes=False)
    kern = functools.partial(
        pl.kernel,
        out_type=jax.ShapeDtypeStruct((b, d), table.dtype),
        mesh=mesh,
        scratch_types=scratch,
        compiler_params=cp,
    )(gather_body)
    if nodes is not None:
        return kern(table, idx, nodes)
    return kern(table, idx)


def _agg_layer(dif, gathered, w, relu, bm, out_packed):
    m_total, s_total = dif.shape
    d = 2 * gathered.shape[1]
    hd = d // 2
    grid = (m_total // bm,)
    dst_block0 = s_total // bm

    def unpack(w_i32):
        wu = jax.lax.bitcast_convert_type(w_i32, jnp.uint32)
        lo = jax.lax.bitcast_convert_type(wu << 16, F32)
        hi = jax.lax.bitcast_convert_type(wu & jnp.uint32(0xFFFF0000), F32)
        return lo.astype(BF16), hi.astype(BF16)

    sh = s_total // 2

    def body(difl_ref, difr_ref, src_ref, dst_ref, wt_ref, wb_ref, o_ref,
             srcb_ref):
        m = pl.program_id(0)

        @pl.when(m == 0)
        def _():
            lo, hi = unpack(src_ref[...])
            srcb_ref[:, :hd] = lo
            srcb_ref[:, hd:] = hi

        agg = (jnp.dot(difl_ref[...].astype(BF16), srcb_ref[pl.ds(0, sh), :],
                       preferred_element_type=F32)
               + jnp.dot(difr_ref[...].astype(BF16),
                         srcb_ref[pl.ds(sh, sh), :],
                         preferred_element_type=F32))
        dlo, dhi = unpack(dst_ref[...])
        o = (jnp.dot(agg.astype(BF16), wt_ref[...].astype(BF16),
                     preferred_element_type=F32)
             + jnp.dot(dlo, wb_ref[pl.ds(0, hd), :].astype(BF16),
                       preferred_element_type=F32)
             + jnp.dot(dhi, wb_ref[pl.ds(hd, hd), :].astype(BF16),
                       preferred_element_type=F32))
        if relu:
            o = jnp.maximum(o, 0.0)
        if out_packed:
            lo = jax.lax.bitcast_convert_type(o[:, :hd], jnp.uint32)
            hi = jax.lax.bitcast_convert_type(o[:, hd:], jnp.uint32)
            packed = (((hi + 0x8000) & jnp.uint32(0xFFFF0000))
                      | ((lo + 0x8000) >> 16))
            o_ref[...] = jax.lax.bitcast_convert_type(packed, jnp.int32)
        else:
            o_ref[...] = o

    out_cols = hd if out_packed else d
    out_dtype = jnp.int32 if out_packed else F32
    return pl.pallas_call(
        body,
        grid=grid,
        in_specs=[
            pl.BlockSpec((bm, sh), lambda m: (m, 0)),
            pl.BlockSpec((bm, sh), lambda m: (m, 1)),
            pl.BlockSpec((s_total, hd), lambda m: (0, 0)),
            pl.BlockSpec((bm, hd), lambda m: (dst_block0 + m, 0)),
            pl.BlockSpec((d, d), lambda m: (0, 0)),
            pl.BlockSpec((d, d), lambda m: (1, 0)),
        ],
        out_specs=pl.BlockSpec((bm, out_cols), lambda m: (m, 0)),
        out_shape=jax.ShapeDtypeStruct((m_total, out_cols), out_dtype),
        scratch_shapes=[pltpu.VMEM((s_total, d), BF16)],
        compiler_params=pltpu.CompilerParams(
            dimension_semantics=("arbitrary",)),
    )(dif, dif, gathered, gathered, w, w)


def _l2_mlp_fused(dif0, dif1, g0, g1, w_agg2, W1, b1, W2, b2, W3, b3, W4,
                  b4, W5, b5, bm):
    m_total, s_total = dif0.shape
    hd = g0.shape[1]
    d = 2 * hd
    nm = m_total // bm
    dst0blk = s_total // bm

    def unpack(w_i32):
        wu = jax.lax.bitcast_convert_type(w_i32, jnp.uint32)
        lo = jax.lax.bitcast_convert_type(wu << 16, F32)
        hi = jax.lax.bitcast_convert_type(wu & jnp.uint32(0xFFFF0000), F32)
        return lo.astype(BF16), hi.astype(BF16)

    def agg_tile(dif_ref, srcb_ref, dst_ref, wt_ref, wb_ref):
        agg = jnp.dot(dif_ref[...].astype(BF16), srcb_ref[...],
                      preferred_element_type=F32)
        dlo, dhi = unpack(dst_ref[...])
        return (jnp.dot(agg.astype(BF16), wt_ref[...].astype(BF16),
                        preferred_element_type=F32)
                + jnp.dot(dlo, wb_ref[pl.ds(0, hd), :].astype(BF16),
                          preferred_element_type=F32)
                + jnp.dot(dhi, wb_ref[pl.ds(hd, hd), :].astype(BF16),
                          preferred_element_type=F32))

    def body(dif0_ref, dif1_ref, src0_ref, src1_ref, dst0_ref, dst1_ref,
             wt_ref, wb_ref, w1t_ref, w1b_ref, b1_ref, w2_ref, b2_ref,
             w3_ref, b3_ref, w4_ref, b4_ref, w5_ref, b5_ref, o_ref,
             srcb0_ref, srcb1_ref, z1_ref):
        b = pl.program_id(0)
        m = pl.program_id(1)

        @pl.when(jnp.logical_and(b == 0, m == 0))
        def _():
            lo, hi = unpack(src0_ref[...])
            srcb0_ref[:, :hd] = lo
            srcb0_ref[:, hd:] = hi

        @pl.when(jnp.logical_and(b == 1, m == 0))
        def _():
            lo, hi = unpack(src1_ref[...])
            srcb1_ref[:, :hd] = lo
            srcb1_ref[:, hd:] = hi

        rows = pl.ds(m * bm, bm)

        @pl.when(b == 0)
        def _():
            h2 = agg_tile(dif0_ref, srcb0_ref, dst0_ref, wt_ref, wb_ref)
            z1_ref[rows, :] = jnp.dot(h2.astype(BF16), w1t_ref[...],
                                      preferred_element_type=F32)

        @pl.when(b == 1)
        def _():
            h2 = agg_tile(dif1_ref, srcb1_ref, dst1_ref, wt_ref, wb_ref)
            z1_ref[rows, :] += jnp.dot(h2.astype(BF16), w1b_ref[...],
                                       preferred_element_type=F32)

        @pl.when(jnp.logical_and(b == 1, m == nm - 1))
        def _():
            z = jnp.maximum(z1_ref[...] + b1_ref[...], 0.0)
            z = jnp.maximum(jnp.dot(z, w2_ref[...],
                                    preferred_element_type=F32)
                            + b2_ref[...], 0.0)
            z = jnp.maximum(jnp.dot(z, w3_ref[...],
                                    preferred_element_type=F32)
                            + b3_ref[...], 0.0)
            z = jnp.maximum(jnp.dot(z, w4_ref[...],
                                    preferred_element_type=F32)
                            + b4_ref[...], 0.0)
            o_ref[...] = (jnp.dot(z, w5_ref[...], preferred_element_type=F32)
                          + b5_ref[...])

    last = nm - 1
    specs = [
        pl.BlockSpec((bm, s_total), lambda b, m: (m + (last - m) * b, 0)),
        pl.BlockSpec((bm, s_total), lambda b, m: (m * b, 0)),
        pl.BlockSpec((s_total, hd), lambda b, m: (0, 0)),
        pl.BlockSpec((s_total, hd), lambda b, m: (0, 0)),
        pl.BlockSpec((bm, hd), lambda b, m: (dst0blk + m + (last - m) * b, 0)),
        pl.BlockSpec((bm, hd), lambda b, m: (dst0blk + m * b, 0)),
        pl.BlockSpec((d, d), lambda b, m: (0, 0)),
        pl.BlockSpec((d, d), lambda b, m: (1, 0)),
        pl.BlockSpec((d, 128), lambda b, m: (0, 0)),
        pl.BlockSpec((d, 128), lambda b, m: (1, 0)),
        pl.BlockSpec((1, 128), lambda b, m: (0, 0)),
        pl.BlockSpec((128, 64), lambda b, m: (0, 0)),
        pl.BlockSpec((1, 64), lambda b, m: (0, 0)),
        pl.BlockSpec((64, 32), lambda b, m: (0, 0)),
        pl.BlockSpec((1, 32), lambda b, m: (0, 0)),
        pl.BlockSpec((32, 8), lambda b, m: (0, 0)),
        pl.BlockSpec((1, 8), lambda b, m: (0, 0)),
        pl.BlockSpec((8, 1), lambda b, m: (0, 0)),
        pl.BlockSpec((1, 1), lambda b, m: (0, 0)),
    ]
    w1b16 = W1.astype(BF16)
    return pl.pallas_call(
        body,
        grid=(2, nm),
        in_specs=specs,
        out_specs=pl.BlockSpec((m_total, 1), lambda b, m: (0, 0)),
        out_shape=jax.ShapeDtypeStruct((m_total, 1), F32),
        scratch_shapes=[pltpu.VMEM((s_total, d), BF16),
                        pltpu.VMEM((s_total, d), BF16),
                        pltpu.VMEM((m_total, 128), F32)],
        compiler_params=pltpu.CompilerParams(
            dimension_semantics=("arbitrary", "arbitrary")),
    )(dif0, dif1, g0, g1, g0, g1, w_agg2, w_agg2, w1b16, w1b16,
      b1.reshape(1, -1), W2, b2.reshape(1, -1), W3, b3.reshape(1, -1),
      W4, b4.reshape(1, -1), W5, b5.reshape(1, -1))


def _pack_halves(x):
    h = x.shape[1] // 2
    lo = jax.lax.bitcast_convert_type(x[:, :h], jnp.uint32)
    hi = jax.lax.bitcast_convert_type(x[:, h:], jnp.uint32)
    packed = (((hi + 0x8000) & jnp.uint32(0xFFFF0000)) | ((lo + 0x8000) >> 16))
    return jax.lax.bitcast_convert_type(packed, jnp.int32)


def _branch_layer1(feats_p, src_nodes, s2s, s2d, dif, w_agg1):
    idx = jnp.concatenate([s2s, s2d])
    g = _sc_gather(feats_p, idx, chunk=128, nodes=src_nodes)
    return _agg_layer(dif, g, w_agg1, relu=True, bm=256, out_packed=True)


def _layer2_gather(h1_p, s2s, s2d):
    idx = jnp.concatenate([s2s, s2d])
    return _sc_gather(h1_p, idx, chunk=32)


def kernel(feats, src_nodes0, dstsrc2src0_1, dstsrc2src0_2, dstsrc2dst0_1,
           dstsrc2dst0_2, dif_mat0_1, dif_mat0_2, src_nodes1, dstsrc2src1_1,
           dstsrc2src1_2, dstsrc2dst1_1, dstsrc2dst1_2, dif_mat1_1,
           dif_mat1_2, w_agg1, w_agg2, W1, b1, W2, b2, W3, b3, W4, b4, W5,
           b5):
    feats_p = _pack_halves(feats)
    h1_0 = _branch_layer1(feats_p, src_nodes0, dstsrc2src0_2, dstsrc2dst0_2,
                          dif_mat0_2, w_agg1)
    h1_1 = _branch_layer1(feats_p, src_nodes1, dstsrc2src1_2, dstsrc2dst1_2,
                          dif_mat1_2, w_agg1)
    g2_0 = _layer2_gather(h1_0, dstsrc2src0_1, dstsrc2dst0_1)
    g2_1 = _layer2_gather(h1_1, dstsrc2src1_1, dstsrc2dst1_1)
    return _l2_mlp_fused(dif_mat0_1, dif_mat1_1, g2_0, g2_1, w_agg2, W1, b1,
                         W2, b2, W3, b3, W4, b4, W5, b5, bm=256)


# speedup vs baseline: 1.5575x; 1.0213x over previous
import dataclasses
import functools

import jax
import jax.numpy as jnp
from jax import lax
from jax.experimental import pallas as pl
from jax.experimental.pallas import tpu as pltpu
from jax.experimental.pallas import tpu_sc as plsc

F32 = jnp.float32
BF16 = jnp.bfloat16

_NC, _NS = 2, 16
_NW = _NC * _NS


def _sc_gather(table, idx, chunk, nodes=None):
    v, d = table.shape
    b = idx.shape[0]
    b_per_w = b // _NW
    assert b % _NW == 0 and b_per_w % chunk == 0 and chunk % 16 == 0
    n_chunks = b_per_w // chunk
    n_nodes = 0 if nodes is None else nodes.shape[0]
    mesh = plsc.VectorSubcoreMesh(core_axis_name="c", subcore_axis_name="s")

    scratch = [
        pltpu.VMEM((3, chunk), jnp.int32),
        pltpu.VMEM((3 * chunk, d), table.dtype),
        pltpu.SemaphoreType.DMA((3,)),
        pltpu.SemaphoreType.DMA((3,)),
    ]
    if nodes is not None:
        scratch.append(pltpu.VMEM((n_nodes,), jnp.int32))

    def gather_body(table_hbm, idx_hbm, *rest):
        if nodes is not None:
            nodes_hbm, out_hbm, idx_v, rows_v, gsem, wsem, nodes_v = rest
            pltpu.sync_copy(nodes_hbm, nodes_v)
        else:
            out_hbm, idx_v, rows_v, gsem, wsem = rest
        wid = lax.axis_index("s") * _NC + lax.axis_index("c")
        base = wid * b_per_w
        gathers = [None, None, None]
        writes = [None, None, None]
        for c in range(n_chunks):
            j = c % 3
            if writes[j] is not None:
                writes[j].wait()
            off = base + c * chunk
            pltpu.sync_copy(idx_hbm.at[pl.ds(off, chunk)], idx_v.at[j])
            if nodes is not None:
                @pl.loop(0, chunk, step=16)
                def _(t, _j=j):
                    raw = idx_v[_j, pl.ds(t, 16)]
                    idx_v[_j, pl.ds(t, 16)] = plsc.load_gather(nodes_v, [raw])
            gathers[j] = pltpu.async_copy(
                table_hbm.at[idx_v.at[j]], rows_v.at[pl.ds(j * chunk, chunk)],
                gsem.at[j])
            if c >= 1:
                pj = (c - 1) % 3
                gathers[pj].wait()
                poff = base + (c - 1) * chunk
                writes[pj] = pltpu.async_copy(
                    rows_v.at[pl.ds(pj * chunk, chunk)],
                    out_hbm.at[pl.ds(poff, chunk)], wsem.at[pj])
        lj = (n_chunks - 1) % 3
        gathers[lj].wait()
        writes[lj] = pltpu.async_copy(
            rows_v.at[pl.ds(lj * chunk, chunk)],
            out_hbm.at[pl.ds(base + (n_chunks - 1) * chunk, chunk)],
            wsem.at[lj])
        for w in writes:
            if w is not None:
                w.wait()

    cp = pltpu.CompilerParams()
    if "needs_layout_passes" in pltpu.CompilerParams.__dataclass_fields__:
        cp = dataclasses.replace(cp, needs_layout_passes=False)
    kern = functools.partial(
        pl.kernel,
        out_type=jax.ShapeDtypeStruct((b, d), table.dtype),
        mesh=mesh,
        scratch_types=scratch,
        compiler_params=cp,
    )(gather_body)
    if nodes is not None:
        return kern(table, idx, nodes)
    return kern(table, idx)


def _agg_layer(dif, gathered, w, relu, bm, out_packed):
    m_total, s_total = dif.shape
    d = 2 * gathered.shape[1]
    hd = d // 2
    grid = (m_total // bm,)
    dst_block0 = s_total // bm

    def unpack(w_i32):
        wu = jax.lax.bitcast_convert_type(w_i32, jnp.uint32)
        lo = jax.lax.bitcast_convert_type(wu << 16, F32)
        hi = jax.lax.bitcast_convert_type(wu & jnp.uint32(0xFFFF0000), F32)
        return lo.astype(BF16), hi.astype(BF16)

    sh = s_total // 2

    def body(difl_ref, difr_ref, src_ref, dst_ref, wt_ref, wb_ref, o_ref,
             srcb_ref):
        m = pl.program_id(0)

        @pl.when(m == 0)
        def _():
            lo, hi = unpack(src_ref[...])
            srcb_ref[:, :hd] = lo
            srcb_ref[:, hd:] = hi

        agg = (jnp.dot(difl_ref[...].astype(BF16), srcb_ref[pl.ds(0, sh), :],
                       preferred_element_type=F32)
               + jnp.dot(difr_ref[...].astype(BF16),
                         srcb_ref[pl.ds(sh, sh), :],
                         preferred_element_type=F32))
        dlo, dhi = unpack(dst_ref[...])
        o = (jnp.dot(agg.astype(BF16), wt_ref[...].astype(BF16),
                     preferred_element_type=F32)
             + jnp.dot(dlo, wb_ref[pl.ds(0, hd), :].astype(BF16),
                       preferred_element_type=F32)
             + jnp.dot(dhi, wb_ref[pl.ds(hd, hd), :].astype(BF16),
                       preferred_element_type=F32))
        if relu:
            o = jnp.maximum(o, 0.0)
        if out_packed:
            lo = jax.lax.bitcast_convert_type(o[:, :hd], jnp.uint32)
            hi = jax.lax.bitcast_convert_type(o[:, hd:], jnp.uint32)
            packed = (((hi + 0x8000) & jnp.uint32(0xFFFF0000))
                      | ((lo + 0x8000) >> 16))
            o_ref[...] = jax.lax.bitcast_convert_type(packed, jnp.int32)
        else:
            o_ref[...] = o

    out_cols = hd if out_packed else d
    out_dtype = jnp.int32 if out_packed else F32
    return pl.pallas_call(
        body,
        grid=grid,
        in_specs=[
            pl.BlockSpec((bm, sh), lambda m: (m, 0)),
            pl.BlockSpec((bm, sh), lambda m: (m, 1)),
            pl.BlockSpec((s_total, hd), lambda m: (0, 0)),
            pl.BlockSpec((bm, hd), lambda m: (dst_block0 + m, 0)),
            pl.BlockSpec((d, d), lambda m: (0, 0)),
            pl.BlockSpec((d, d), lambda m: (1, 0)),
        ],
        out_specs=pl.BlockSpec((bm, out_cols), lambda m: (m, 0)),
        out_shape=jax.ShapeDtypeStruct((m_total, out_cols), out_dtype),
        scratch_shapes=[pltpu.VMEM((s_total, d), BF16)],
        compiler_params=pltpu.CompilerParams(
            dimension_semantics=("arbitrary",)),
    )(dif, dif, gathered, gathered, w, w)


def _l2_mlp_fused(dif0, dif1, g0, g1, w_agg2, W1, b1, W2, b2, W3, b3, W4,
                  b4, W5, b5, bm):
    m_total, s_total = dif0.shape
    hd = g0.shape[1]
    d = 2 * hd
    nm = m_total // bm
    dst0blk = s_total // bm

    def unpack(w_i32):
        wu = jax.lax.bitcast_convert_type(w_i32, jnp.uint32)
        lo = jax.lax.bitcast_convert_type(wu << 16, F32)
        hi = jax.lax.bitcast_convert_type(wu & jnp.uint32(0xFFFF0000), F32)
        return lo.astype(BF16), hi.astype(BF16)

    def agg_tile(dif_ref, srcb_ref, dst_ref, wt_ref, wb_ref):
        agg = jnp.dot(dif_ref[...].astype(BF16), srcb_ref[...],
                      preferred_element_type=F32)
        dlo, dhi = unpack(dst_ref[...])
        return (jnp.dot(agg.astype(BF16), wt_ref[...].astype(BF16),
                        preferred_element_type=F32)
                + jnp.dot(dlo, wb_ref[pl.ds(0, hd), :].astype(BF16),
                          preferred_element_type=F32)
                + jnp.dot(dhi, wb_ref[pl.ds(hd, hd), :].astype(BF16),
                          preferred_element_type=F32))

    def body(dif0_ref, dif1_ref, src0_ref, src1_ref, dst0_ref, dst1_ref,
             wt_ref, wb_ref, w1t_ref, w1b_ref, b1_ref, w2_ref, b2_ref,
             w3_ref, b3_ref, w4_ref, b4_ref, w5_ref, b5_ref, o_ref,
             srcb0_ref, srcb1_ref, z1_ref):
        b = pl.program_id(0)
        m = pl.program_id(1)

        @pl.when(jnp.logical_and(b == 0, m == 0))
        def _():
            lo, hi = unpack(src0_ref[...])
            srcb0_ref[:, :hd] = lo
            srcb0_ref[:, hd:] = hi

        @pl.when(jnp.logical_and(b == 1, m == 0))
        def _():
            lo, hi = unpack(src1_ref[...])
            srcb1_ref[:, :hd] = lo
            srcb1_ref[:, hd:] = hi

        rows = pl.ds(m * bm, bm)

        @pl.when(b == 0)
        def _():
            h2 = agg_tile(dif0_ref, srcb0_ref, dst0_ref, wt_ref, wb_ref)
            z1_ref[rows, :] = jnp.dot(h2.astype(BF16), w1t_ref[...],
                                      preferred_element_type=F32)

        @pl.when(b == 1)
        def _():
            h2 = agg_tile(dif1_ref, srcb1_ref, dst1_ref, wt_ref, wb_ref)
            z1_ref[rows, :] += jnp.dot(h2.astype(BF16), w1b_ref[...],
                                       preferred_element_type=F32)

        @pl.when(jnp.logical_and(b == 1, m == nm - 1))
        def _():
            z = jnp.maximum(z1_ref[...] + b1_ref[...], 0.0)
            z = jnp.maximum(jnp.dot(z, w2_ref[...],
                                    preferred_element_type=F32)
                            + b2_ref[...], 0.0)
            z = jnp.maximum(jnp.dot(z, w3_ref[...],
                                    preferred_element_type=F32)
                            + b3_ref[...], 0.0)
            z = jnp.maximum(jnp.dot(z, w4_ref[...],
                                    preferred_element_type=F32)
                            + b4_ref[...], 0.0)
            o_ref[...] = (jnp.dot(z, w5_ref[...], preferred_element_type=F32)
                          + b5_ref[...])

    last = nm - 1
    specs = [
        pl.BlockSpec((bm, s_total), lambda b, m: (m + (last - m) * b, 0)),
        pl.BlockSpec((bm, s_total), lambda b, m: (m * b, 0)),
        pl.BlockSpec((s_total, hd), lambda b, m: (0, 0)),
        pl.BlockSpec((s_total, hd), lambda b, m: (0, 0)),
        pl.BlockSpec((bm, hd), lambda b, m: (dst0blk + m + (last - m) * b, 0)),
        pl.BlockSpec((bm, hd), lambda b, m: (dst0blk + m * b, 0)),
        pl.BlockSpec((d, d), lambda b, m: (0, 0)),
        pl.BlockSpec((d, d), lambda b, m: (1, 0)),
        pl.BlockSpec((d, 128), lambda b, m: (0, 0)),
        pl.BlockSpec((d, 128), lambda b, m: (1, 0)),
        pl.BlockSpec((1, 128), lambda b, m: (0, 0)),
        pl.BlockSpec((128, 64), lambda b, m: (0, 0)),
        pl.BlockSpec((1, 64), lambda b, m: (0, 0)),
        pl.BlockSpec((64, 32), lambda b, m: (0, 0)),
        pl.BlockSpec((1, 32), lambda b, m: (0, 0)),
        pl.BlockSpec((32, 8), lambda b, m: (0, 0)),
        pl.BlockSpec((1, 8), lambda b, m: (0, 0)),
        pl.BlockSpec((8, 1), lambda b, m: (0, 0)),
        pl.BlockSpec((1, 1), lambda b, m: (0, 0)),
    ]
    w1b16 = W1.astype(BF16)
    return pl.pallas_call(
        body,
        grid=(2, nm),
        in_specs=specs,
        out_specs=pl.BlockSpec((m_total, 1), lambda b, m: (0, 0)),
        out_shape=jax.ShapeDtypeStruct((m_total, 1), F32),
        scratch_shapes=[pltpu.VMEM((s_total, d), BF16),
                        pltpu.VMEM((s_total, d), BF16),
                        pltpu.VMEM((m_total, 128), F32)],
        compiler_params=pltpu.CompilerParams(
            dimension_semantics=("arbitrary", "arbitrary")),
    )(dif0, dif1, g0, g1, g0, g1, w_agg2, w_agg2, w1b16, w1b16,
      b1.reshape(1, -1), W2, b2.reshape(1, -1), W3, b3.reshape(1, -1),
      W4, b4.reshape(1, -1), W5, b5.reshape(1, -1))


def _pack_halves(x):
    h = x.shape[1] // 2
    lo = jax.lax.bitcast_convert_type(x[:, :h], jnp.uint32)
    hi = jax.lax.bitcast_convert_type(x[:, h:], jnp.uint32)
    packed = (((hi + 0x8000) & jnp.uint32(0xFFFF0000)) | ((lo + 0x8000) >> 16))
    return jax.lax.bitcast_convert_type(packed, jnp.int32)


def _branch_layer1(feats_p, src_nodes, s2s, s2d, dif, w_agg1):
    idx = jnp.concatenate([s2s, s2d])
    g = _sc_gather(feats_p, idx, chunk=128, nodes=src_nodes)
    return _agg_layer(dif, g, w_agg1, relu=True, bm=512, out_packed=True)


def _layer2_gather(h1_p, s2s, s2d):
    idx = jnp.concatenate([s2s, s2d])
    return _sc_gather(h1_p, idx, chunk=32)


def kernel(feats, src_nodes0, dstsrc2src0_1, dstsrc2src0_2, dstsrc2dst0_1,
           dstsrc2dst0_2, dif_mat0_1, dif_mat0_2, src_nodes1, dstsrc2src1_1,
           dstsrc2src1_2, dstsrc2dst1_1, dstsrc2dst1_2, dif_mat1_1,
           dif_mat1_2, w_agg1, w_agg2, W1, b1, W2, b2, W3, b3, W4, b4, W5,
           b5):
    feats_p = _pack_halves(feats)
    h1_0 = _branch_layer1(feats_p, src_nodes0, dstsrc2src0_2, dstsrc2dst0_2,
                          dif_mat0_2, w_agg1)
    h1_1 = _branch_layer1(feats_p, src_nodes1, dstsrc2src1_2, dstsrc2dst1_2,
                          dif_mat1_2, w_agg1)
    g2_0 = _layer2_gather(h1_0, dstsrc2src0_1, dstsrc2dst0_1)
    g2_1 = _layer2_gather(h1_1, dstsrc2src1_1, dstsrc2dst1_1)
    return _l2_mlp_fused(dif_mat0_1, dif_mat1_1, g2_0, g2_1, w_agg2, W1, b1,
                         W2, b2, W3, b3, W4, b4, W5, b5, bm=256)
